# baseline (speedup 1.0000x reference)
import functools

import jax
import jax.numpy as jnp
from jax import lax
from jax.experimental import pallas as pl
from jax.experimental.pallas import tpu as pltpu
from jax.experimental.pallas import tpu_sc as plsc

_NC = 2
_NS = 16
_NW = _NC * _NS
_R = 8
_L = 16
_U = 8


def _make_sc_add(B, S, D, dtype):
    rows_per_w = S // _NW
    n_chunks = rows_per_w // _R
    cw = _R * D
    n_iters = cw // (_L * _U)

    @functools.partial(
        pl.kernel,
        mesh=plsc.VectorSubcoreMesh(core_axis_name="c", subcore_axis_name="s"),
        out_type=jax.ShapeDtypeStruct((B * S * D,), dtype),
        scratch_types=[
            pltpu.VMEM((2, B, cw), dtype),
            pltpu.VMEM((2, cw), dtype),
            pltpu.SemaphoreType.DMA,
            pltpu.SemaphoreType.DMA,
            pltpu.SemaphoreType.DMA,
            pltpu.SemaphoreType.DMA,
            pltpu.SemaphoreType.DMA,
            pltpu.SemaphoreType.DMA,
        ],
    )
    def sc_add(x_hbm, pos_hbm, out_hbm, xb, pb, ls0, ls1, ss0, ss1, ps0, ps1):
        ls = (ls0, ls1)
        ss = (ss0, ss1)
        ps = (ps0, ps1)
        wid = lax.axis_index("s") * _NC + lax.axis_index("c")
        base = wid * rows_per_w

        def chunk_off(c):
            return (base + c * _R) * D

        def x_copy(c, b, bank, sem):
            xoff = b * S * D + chunk_off(c)
            return pltpu.make_async_copy(
                x_hbm.at[pl.ds(xoff, cw)], xb.at[bank, b], sem)

        def out_copy(c, b, bank, sem):
            xoff = b * S * D + chunk_off(c)
            return pltpu.make_async_copy(
                xb.at[bank, b], out_hbm.at[pl.ds(xoff, cw)], sem)

        def pos_copy(c, bank, sem):
            return pltpu.make_async_copy(
                pos_hbm.at[pl.ds(chunk_off(c), cw)], pb.at[bank], sem)

        pos_copy(0, 0, ps[0]).start()
        for b in range(B):
            x_copy(0, b, 0, ls[0]).start()

        def process(g, bank):
            nxt = jnp.minimum(g + 1, n_chunks - 1)
            other = 1 - bank
            @pl.when(g > 0)
            def _():
                for b in range(B):
                    out_copy(g - 1, b, other, ss[other]).wait()
            for b in range(B):
                x_copy(nxt, b, other, ls[other]).start()
            pos_copy(g, bank, ps[bank]).wait()
            pos_copy(nxt, other, ps[other]).start()
            for b in range(B):
                x_copy(g, b, bank, ls[bank]).wait()

                def add_body(k, c):
                    kb = k * (_L * _U)
                    for u in range(_U):
                        sl = pl.ds(kb + u * _L, _L)
                        plsc.addupdate(xb.at[bank, b, sl], pb[bank, sl])
                    return c

                lax.fori_loop(0, n_iters, add_body, 0)
                out_copy(g, b, bank, ss[bank]).start()

        def two_chunks(gg, carry):
            process(2 * gg, 0)
            process(2 * gg + 1, 1)
            return carry

        lax.fori_loop(0, n_chunks // 2, two_chunks, 0)

        last = n_chunks - 1
        for b in range(B):
            out_copy(last, b, 1, ss[1]).wait()
            x_copy(last, b, 0, ls[0]).wait()
        pos_copy(last, 0, ps[0]).wait()

    return sc_add


def kernel(x, pos_table):
    B, S, D = x.shape
    out = _make_sc_add(B, S, D, x.dtype)(
        x.reshape(B * S * D), pos_table.reshape(S * D)
    )
    return out.reshape(B, S, D)


# speedup vs baseline: 1.3412x; 1.3412x over previous
import functools

import jax
import jax.numpy as jnp
from jax import lax
from jax.experimental import pallas as pl
from jax.experimental.pallas import tpu as pltpu
from jax.experimental.pallas import tpu_sc as plsc

_NC = 2
_NS = 16
_NW = _NC * _NS
_R = 8
_L = 16
_U = 8


def _make_sc_add(B, S, D, dtype):
    rows_per_w = S // _NW
    n_chunks = rows_per_w // _R
    cw = _R * D
    n_iters = cw // (_L * _U)

    @functools.partial(
        pl.kernel,
        mesh=plsc.VectorSubcoreMesh(core_axis_name="c", subcore_axis_name="s"),
        out_type=jax.ShapeDtypeStruct((B * S * D,), dtype),
        scratch_types=[
            pltpu.VMEM((2, B, cw), dtype),
            pltpu.VMEM((2, cw), dtype),
            pltpu.SemaphoreType.DMA,
            pltpu.SemaphoreType.DMA,
            pltpu.SemaphoreType.DMA,
            pltpu.SemaphoreType.DMA,
            pltpu.SemaphoreType.DMA,
            pltpu.SemaphoreType.DMA,
        ],
    )
    def sc_add(x_hbm, pos_hbm, out_hbm, xb, pb, ls0, ls1, ss0, ss1, ps0, ps1):
        ls = (ls0, ls1)
        ss = (ss0, ss1)
        ps = (ps0, ps1)
        wid = lax.axis_index("s") * _NC + lax.axis_index("c")
        base = wid * rows_per_w

        def chunk_off(c):
            return (base + c * _R) * D

        def x_copy(c, b, bank, sem):
            xoff = b * S * D + chunk_off(c)
            return pltpu.make_async_copy(
                x_hbm.at[pl.ds(xoff, cw)], xb.at[bank, b], sem)

        def out_copy(c, b, bank, sem):
            xoff = b * S * D + chunk_off(c)
            return pltpu.make_async_copy(
                xb.at[bank, b], out_hbm.at[pl.ds(xoff, cw)], sem)

        def pos_copy(c, bank, sem):
            return pltpu.make_async_copy(
                pos_hbm.at[pl.ds(chunk_off(c), cw)], pb.at[bank], sem)

        pos_copy(0, 0, ps[0]).start()
        for b in range(B):
            x_copy(0, b, 0, ls[0]).start()

        def process(g, bank):
            nxt = jnp.minimum(g + 1, n_chunks - 1)
            other = 1 - bank
            @pl.when(g > 0)
            def _():
                for b in range(B):
                    out_copy(g - 1, b, other, ss[other]).wait()
            for b in range(B):
                x_copy(nxt, b, other, ls[other]).start()
            pos_copy(g, bank, ps[bank]).wait()
            pos_copy(nxt, other, ps[other]).start()
            for b in range(B):
                x_copy(g, b, bank, ls[bank]).wait()

                def add_body(k, c):
                    kb = k * (_L * _U)
                    sls = [pl.ds(kb + u * _L, _L) for u in range(_U)]
                    vals = [pb[bank, sl] for sl in sls]
                    for sl, v in zip(sls, vals):
                        plsc.addupdate(xb.at[bank, b, sl], v)
                    return c

                lax.fori_loop(0, n_iters, add_body, 0)
                out_copy(g, b, bank, ss[bank]).start()

        def two_chunks(gg, carry):
            process(2 * gg, 0)
            process(2 * gg + 1, 1)
            return carry

        lax.fori_loop(0, n_chunks // 2, two_chunks, 0)

        last = n_chunks - 1
        for b in range(B):
            out_copy(last, b, 1, ss[1]).wait()
            x_copy(last, b, 0, ls[0]).wait()
        pos_copy(last, 0, ps[0]).wait()

    return sc_add


def kernel(x, pos_table):
    B, S, D = x.shape
    out = _make_sc_add(B, S, D, x.dtype)(
        x.reshape(B * S * D), pos_table.reshape(S * D)
    )
    return out.reshape(B, S, D)


# speedup vs baseline: 4.8404x; 3.6089x over previous
import functools

import jax
import jax.numpy as jnp
from jax import lax
from jax.experimental import pallas as pl
from jax.experimental.pallas import tpu as pltpu
from jax.experimental.pallas import tpu_sc as plsc

_NC = 2
_NS = 16
_NW = _NC * _NS
_R = 8
_L = 16
_U = 8


def _make_sc_add(B, S, D, dtype):
    rows_per_w = S // _NW
    n_chunks = rows_per_w // _R
    n_iters = _R * D // (_L * _U)

    @functools.partial(
        pl.kernel,
        mesh=plsc.VectorSubcoreMesh(core_axis_name="c", subcore_axis_name="s"),
        out_type=jax.ShapeDtypeStruct((B * S, D), dtype),
        scratch_types=[
            pltpu.VMEM((2, B, _R, D), dtype),
            pltpu.VMEM((2, _R, D), dtype),
            pltpu.SemaphoreType.DMA,
            pltpu.SemaphoreType.DMA,
            pltpu.SemaphoreType.DMA,
            pltpu.SemaphoreType.DMA,
            pltpu.SemaphoreType.DMA,
            pltpu.SemaphoreType.DMA,
        ],
    )
    def sc_add(x_hbm, pos_hbm, out_hbm, xb, pb, ls0, ls1, ss0, ss1, ps0, ps1):
        ls = (ls0, ls1)
        ss = (ss0, ss1)
        ps = (ps0, ps1)
        wid = lax.axis_index("s") * _NC + lax.axis_index("c")
        base = wid * rows_per_w

        def x_copy(c, b, bank, sem):
            row = b * S + base + c * _R
            return pltpu.make_async_copy(
                x_hbm.at[pl.ds(row, _R)], xb.at[bank, b], sem)

        def out_copy(c, b, bank, sem):
            row = b * S + base + c * _R
            return pltpu.make_async_copy(
                xb.at[bank, b], out_hbm.at[pl.ds(row, _R)], sem)

        def pos_copy(c, bank, sem):
            row = base + c * _R
            return pltpu.make_async_copy(
                pos_hbm.at[pl.ds(row, _R)], pb.at[bank], sem)

        pos_copy(0, 0, ps[0]).start()
        for b in range(B):
            x_copy(0, b, 0, ls[0]).start()

        def process(g, bank):
            nxt = jnp.minimum(g + 1, n_chunks - 1)
            other = 1 - bank
            @pl.when(g > 0)
            def _():
                for b in range(B):
                    out_copy(g - 1, b, other, ss[other]).wait()
            for b in range(B):
                x_copy(nxt, b, other, ls[other]).start()
            pos_copy(g, bank, ps[bank]).wait()
            pos_copy(nxt, other, ps[other]).start()
            for b in range(B):
                x_copy(g, b, bank, ls[bank]).wait()

                def add_body(k, c):
                    r = k >> 3
                    col = (k & 7) * (_L * _U)
                    sls = [pl.ds(col + u * _L, _L) for u in range(_U)]
                    vals = [pb[bank, r, sl] for sl in sls]
                    for sl, v in zip(sls, vals):
                        plsc.addupdate(xb.at[bank, b, r, sl], v)
                    return c

                lax.fori_loop(0, n_iters, add_body, 0)
                out_copy(g, b, bank, ss[bank]).start()

        def two_chunks(gg, carry):
            process(2 * gg, 0)
            process(2 * gg + 1, 1)
            return carry

        lax.fori_loop(0, n_chunks // 2, two_chunks, 0)

        last = n_chunks - 1
        for b in range(B):
            out_copy(last, b, 1, ss[1]).wait()
            x_copy(last, b, 0, ls[0]).wait()
        pos_copy(last, 0, ps[0]).wait()

    return sc_add


def kernel(x, pos_table):
    B, S, D = x.shape
    out = _make_sc_add(B, S, D, x.dtype)(x.reshape(B * S, D), pos_table)
    return out.reshape(B, S, D)
